# baseline (speedup 1.0000x reference)
import functools

import jax
import jax.numpy as jnp
from jax.experimental import pallas as pl
from jax.experimental.pallas import tpu as pltpu

_TILE_B = 65536


def _mlp_body(x_ref, w0_ref, b0_ref, w1_ref, b1_ref, w2_ref, b2_ref, o_ref):
    bf = jnp.bfloat16
    x16 = x_ref[...].astype(bf)
    h = jnp.dot(w0_ref[...].astype(bf), x16, preferred_element_type=jnp.float32)
    h = jnp.maximum(h.astype(bf) + b0_ref[...].astype(bf), 0)
    h = jnp.dot(w1_ref[...].astype(bf), h, preferred_element_type=jnp.float32)
    h = jnp.maximum(h.astype(bf) + b1_ref[...].astype(bf), 0)
    out = jnp.dot(w2_ref[...].astype(bf), h, preferred_element_type=jnp.float32)
    o_ref[...] = out + b2_ref[...]


@jax.jit
def _forward(x, w0, b0, w1, b1, w2, b2):
    B, in_f = x.shape

    x_t = x.T

    num_tiles = pl.cdiv(B, _TILE_B)
    tile_b = min(_TILE_B, ((B + num_tiles * 128 - 1) // (num_tiles * 128)) * 128)
    padded_b = num_tiles * tile_b
    if padded_b != B:
        x_t = jnp.pad(x_t, ((0, 0), (0, padded_b - B)))

    const_map = lambda i: (0, 0)
    out = pl.pallas_call(
        _mlp_body,
        out_shape=jax.ShapeDtypeStruct((1, padded_b), jnp.float32),
        grid=(num_tiles,),
        in_specs=[
            pl.BlockSpec((in_f, tile_b), lambda i: (0, i)),
            pl.BlockSpec((7, in_f), const_map),
            pl.BlockSpec((7, 1), const_map),
            pl.BlockSpec((7, 7), const_map),
            pl.BlockSpec((7, 1), const_map),
            pl.BlockSpec((1, 7), const_map),
            pl.BlockSpec((1, 1), const_map),
        ],
        out_specs=pl.BlockSpec((1, tile_b), lambda i: (0, i)),
        compiler_params=pltpu.CompilerParams(
            dimension_semantics=("parallel",),
        ),
    )(x_t, w0, b0, w1, b1, w2, b2)

    return out[:, :B].T


def kernel(x, w0, b0, w1, b1, w2, b2):
    return _forward(x, w0, b0, w1, b1, w2, b2)


# speedup vs baseline: 1.0484x; 1.0484x over previous
import functools

import jax
import jax.numpy as jnp
from jax.experimental import pallas as pl
from jax.experimental.pallas import tpu as pltpu

_TILE_B = 262144


def _mlp_body(x_ref, w0_ref, b0_ref, w1_ref, b1_ref, w2_ref, b2_ref, o_ref):
    bf = jnp.bfloat16
    x16 = x_ref[...].astype(bf)
    h = jnp.dot(w0_ref[...].astype(bf), x16, preferred_element_type=jnp.float32)
    h = jnp.maximum(h.astype(bf) + b0_ref[...].astype(bf), 0)
    h = jnp.dot(w1_ref[...].astype(bf), h, preferred_element_type=jnp.float32)
    h = jnp.maximum(h.astype(bf) + b1_ref[...].astype(bf), 0)
    out = jnp.dot(w2_ref[...].astype(bf), h, preferred_element_type=jnp.float32)
    o_ref[...] = out + b2_ref[...]


@jax.jit
def _forward(x, w0, b0, w1, b1, w2, b2):
    B, in_f = x.shape

    x_t = x.T

    num_tiles = pl.cdiv(B, _TILE_B)
    tile_b = min(_TILE_B, ((B + num_tiles * 128 - 1) // (num_tiles * 128)) * 128)
    padded_b = num_tiles * tile_b
    if padded_b != B:
        x_t = jnp.pad(x_t, ((0, 0), (0, padded_b - B)))

    const_map = lambda i: (0, 0)
    out = pl.pallas_call(
        _mlp_body,
        out_shape=jax.ShapeDtypeStruct((1, padded_b), jnp.float32),
        grid=(num_tiles,),
        in_specs=[
            pl.BlockSpec((in_f, tile_b), lambda i: (0, i)),
            pl.BlockSpec((7, in_f), const_map),
            pl.BlockSpec((7, 1), const_map),
            pl.BlockSpec((7, 7), const_map),
            pl.BlockSpec((7, 1), const_map),
            pl.BlockSpec((1, 7), const_map),
            pl.BlockSpec((1, 1), const_map),
        ],
        out_specs=pl.BlockSpec((1, tile_b), lambda i: (0, i)),
        compiler_params=pltpu.CompilerParams(
            dimension_semantics=("parallel",),
        ),
    )(x_t, w0, b0, w1, b1, w2, b2)

    return out[:, :B].T


def kernel(x, w0, b0, w1, b1, w2, b2):
    return _forward(x, w0, b0, w1, b1, w2, b2)
